# Initial kernel scaffold; baseline (speedup 1.0000x reference)
import jax
import jax.numpy as jnp
from jax.experimental import pallas as pl


def kernel(x, edge_index, W_mp, b_mp, W_ne, W_e, W_en, W_recon):
    raise NotImplementedError("write your pallas kernel here")


# speedup vs baseline: 11.3837x; 11.3837x over previous
import functools

import jax
import jax.numpy as jnp
from jax import lax
from jax.experimental import pallas as pl
from jax.experimental.pallas import tpu as pltpu
from jax.experimental.pallas import tpu_sc as plsc

N = 10000
NP = 10240
E = 320000
NC = 2
NS = 16
NW = NC * NS
EPT = E // NW
CH = (EPT + 127) // 128
EPTP = CH * 128
PT = NP // NS

def _wid():
    return lax.axis_index("s") * NC + lax.axis_index("c")


def _hist_body(eir_hbm, eic_hbm, z_hbm, out_hbm, idxr, idxc, ones_v, acc_c, acc_r):
    cid = lax.axis_index("c")
    sid = lax.axis_index("s")
    wid = _wid()
    pltpu.sync_copy(z_hbm.at[pl.ds(sid * PT, PT)], acc_c.at[pl.ds(sid * PT, PT)])
    pltpu.sync_copy(z_hbm.at[pl.ds(sid * PT, PT)], acc_r.at[pl.ds(sid * PT, PT)])
    for i in range(8):
        ones_v[pl.ds(i * 16, 16)] = jnp.full((16,), 1.0, jnp.float32)
    pltpu.sync_copy(eir_hbm.at[wid], idxr)
    pltpu.sync_copy(eic_hbm.at[wid], idxc)
    plsc.subcore_barrier()

    def body(j, carry):
        pltpu.sync_copy(ones_v, acc_c.at[idxc.at[j]], add=True)
        pltpu.sync_copy(ones_v, acc_r.at[idxr.at[j]], add=True)
        return carry

    lax.fori_loop(0, CH, body, 0)
    plsc.subcore_barrier()
    pltpu.sync_copy(acc_c.at[pl.ds(sid * PT, PT)], out_hbm.at[cid, 0, pl.ds(sid * PT, PT)])
    pltpu.sync_copy(acc_r.at[pl.ds(sid * PT, PT)], out_hbm.at[cid, 1, pl.ds(sid * PT, PT)])


def _segsum_body(hd_hbm, eir_hbm, eic_hbm, z2_hbm, out_hbm, idxr, idxc, rows, s_sh, sem):
    cid = lax.axis_index("c")
    sid = lax.axis_index("s")
    wid = _wid()
    pltpu.sync_copy(z2_hbm.at[pl.ds(sid * PT, PT)], s_sh.at[pl.ds(sid * PT, PT)])
    pltpu.sync_copy(eir_hbm.at[wid], idxr)
    pltpu.sync_copy(eic_hbm.at[wid], idxc)
    plsc.subcore_barrier()

    def body(j, carry):
        pltpu.async_copy(hd_hbm.at[idxr.at[j]], rows, sem).wait()
        pltpu.sync_copy(rows, s_sh.at[idxc.at[j]], add=True)
        return carry

    lax.fori_loop(0, CH, body, 0)
    plsc.subcore_barrier()
    pltpu.sync_copy(s_sh.at[pl.ds(sid * PT, PT)], out_hbm.at[cid, pl.ds(sid * PT, PT)])


def _egather_body(g_hbm, eir_hbm, out_hbm, idxr, rows, rows64, sem):
    wid = _wid()
    base = wid * EPT
    pltpu.sync_copy(eir_hbm.at[wid], idxr)
    nfull = EPT // 128

    def compact(nrows):
        def crow(rr, carry):
            for kk in range(4):
                rows64[rr, pl.ds(kk * 16, 16)] = rows[rr, pl.ds(kk * 16, 16)]
            return carry
        lax.fori_loop(0, nrows, crow, 0)

    def body(j, carry):
        pltpu.async_copy(g_hbm.at[idxr.at[j]], rows, sem).wait()
        compact(128)
        pltpu.sync_copy(rows64, out_hbm.at[pl.ds(base + j * 128, 128)])
        return carry

    lax.fori_loop(0, nfull, body, 0)
    tail = EPT - nfull * 128
    pltpu.async_copy(g_hbm.at[idxr.at[nfull]], rows, sem).wait()
    compact(tail)
    pltpu.sync_copy(rows64.at[pl.ds(0, tail)], out_hbm.at[pl.ds(base + nfull * 128, tail)])


@functools.lru_cache(maxsize=None)
def _sc_kernels():
    mesh = plsc.VectorSubcoreMesh(core_axis_name="c", subcore_axis_name="s",
                                  num_cores=NC, num_subcores=NS)
    hist = pl.kernel(
        _hist_body,
        out_type=jax.ShapeDtypeStruct((NC, 2, NP), jnp.float32),
        mesh=mesh,
        scratch_types=[
            pltpu.VMEM((CH, 128), jnp.int32),
            pltpu.VMEM((CH, 128), jnp.int32),
            pltpu.VMEM((128,), jnp.float32),
            pltpu.VMEM_SHARED((NP,), jnp.float32),
            pltpu.VMEM_SHARED((NP,), jnp.float32),
        ],
    )
    segsum = pl.kernel(
        _segsum_body,
        out_type=jax.ShapeDtypeStruct((NC, NP, 128), jnp.float32),
        mesh=mesh,
        scratch_types=[
            pltpu.VMEM((CH, 128), jnp.int32),
            pltpu.VMEM((CH, 128), jnp.int32),
            pltpu.VMEM((128, 128), jnp.float32),
            pltpu.VMEM_SHARED((NP, 128), jnp.float32),
            pltpu.SemaphoreType.DMA,
        ],
    )
    egather = pl.kernel(
        _egather_body,
        out_type=jax.ShapeDtypeStruct((E, 64), jnp.float32),
        mesh=mesh,
        scratch_types=[
            pltpu.VMEM((CH, 128), jnp.int32),
            pltpu.VMEM((128, 128), jnp.float32),
            pltpu.VMEM((128, 64), jnp.float32),
            pltpu.SemaphoreType.DMA,
        ],
    )
    return hist, segsum, egather


def _dense1_body(x_ref, w_ref, hist_ref, hd_ref, hod_ref):
    h = lax.dot_general(x_ref[...], w_ref[...], (((1,), (1,)), ((), ())),
                        preferred_element_type=jnp.float32)
    cnt = hist_ref[0, 0, :] + hist_ref[1, 0, :]
    dis = lax.rsqrt(cnt + 1.0).reshape(NP, 1)
    hd_ref[...] = h * dis
    hod_ref[...] = h * (dis * dis)


def _dense1(xp, W_mp, hist):
    return pl.pallas_call(
        _dense1_body,
        out_shape=[jax.ShapeDtypeStruct((NP, 128), jnp.float32),
                   jax.ShapeDtypeStruct((NP, 128), jnp.float32)],
    )(xp, W_mp, hist)


def _dense2_body(s_ref, hod_ref, hist_ref, b_ref, wne_ref, we_ref, wen_ref,
                 wrec_ref, g_ref, xr_ref):
    cnt = hist_ref[0, 0, :] + hist_ref[1, 0, :]
    dis = lax.rsqrt(cnt + 1.0).reshape(NP, 1)
    odeg = (hist_ref[0, 1, :] + hist_ref[1, 1, :]).reshape(NP, 1)
    s = s_ref[0] + s_ref[1]
    h_agg = s * dis + hod_ref[...] + b_ref[...]
    w1 = lax.dot_general(we_ref[...], wne_ref[...], (((1,), (0,)), ((), ())),
                         preferred_element_type=jnp.float32)
    g = lax.dot_general(h_agg, w1, (((1,), (1,)), ((), ())),
                        preferred_element_type=jnp.float32)
    g_ref[...] = jnp.concatenate([g, jnp.zeros_like(g)], axis=1)
    wc2 = lax.dot_general(wen_ref[...], w1, (((1,), (0,)), ((), ())),
                          preferred_element_type=jnp.float32)
    w3 = lax.dot_general(wrec_ref[...], wc2, (((1,), (0,)), ((), ())),
                         preferred_element_type=jnp.float32)
    xr_ref[...] = lax.dot_general(h_agg, w3, (((1,), (1,)), ((), ())),
                                  preferred_element_type=jnp.float32) * odeg


def _dense2(s_part, hod, hist, b2, W_ne, W_e, W_en, W_recon):
    return pl.pallas_call(
        _dense2_body,
        out_shape=[jax.ShapeDtypeStruct((NP, 128), jnp.float32),
                   jax.ShapeDtypeStruct((NP, 128), jnp.float32)],
    )(s_part, hod, hist, b2, W_ne, W_e, W_en, W_recon)


def kernel(x, edge_index, W_mp, b_mp, W_ne, W_e, W_en, W_recon):
    n = x.shape[0]
    xp = jnp.pad(x, ((0, NP - n), (0, 0)))
    ei = jnp.pad(edge_index.reshape(2, NW, EPT), ((0, 0), (0, 0), (0, EPTP - EPT)),
                 constant_values=N)
    eir3 = ei[0].reshape(NW, CH, 128)
    eic3 = ei[1].reshape(NW, CH, 128)
    z1 = jnp.zeros((NP,), jnp.float32)
    z2 = jnp.zeros((NP, 128), jnp.float32)

    hist_k, segsum_k, egather_k = _sc_kernels()
    hist = hist_k(eir3, eic3, z1)
    hd, hod = _dense1(xp, W_mp, hist)
    s_part = segsum_k(hd, eir3, eic3, z2)
    g, xr = _dense2(s_part, hod, hist, b_mp.reshape(1, -1), W_ne, W_e, W_en,
                    W_recon)
    ef = egather_k(g, eir3)
    return (xr[:n], ef)


# speedup vs baseline: 13.7288x; 1.2060x over previous
import functools

import jax
import jax.numpy as jnp
from jax import lax
from jax.experimental import pallas as pl
from jax.experimental.pallas import tpu as pltpu
from jax.experimental.pallas import tpu_sc as plsc

N = 10000
NP = 10240
E = 320000
NC = 2
NS = 16
NW = NC * NS
EPT = E // NW
CH = (EPT + 127) // 128
EPTP = CH * 128
PT = NP // NS

def _wid():
    return lax.axis_index("s") * NC + lax.axis_index("c")


def _hist_body(eir_hbm, eic_hbm, z_hbm, out_hbm, idxr, idxc, ones_v, acc_c, acc_r):
    cid = lax.axis_index("c")
    sid = lax.axis_index("s")
    wid = _wid()
    pltpu.sync_copy(z_hbm.at[pl.ds(sid * PT, PT)], acc_c.at[pl.ds(sid * PT, PT)])
    pltpu.sync_copy(z_hbm.at[pl.ds(sid * PT, PT)], acc_r.at[pl.ds(sid * PT, PT)])
    for i in range(8):
        ones_v[pl.ds(i * 16, 16)] = jnp.full((16,), 1.0, jnp.float32)
    pltpu.sync_copy(eir_hbm.at[wid], idxr)
    pltpu.sync_copy(eic_hbm.at[wid], idxc)
    plsc.subcore_barrier()

    def body(j, carry):
        pltpu.sync_copy(ones_v, acc_c.at[idxc.at[j]], add=True)
        pltpu.sync_copy(ones_v, acc_r.at[idxr.at[j]], add=True)
        return carry

    lax.fori_loop(0, CH, body, 0)
    plsc.subcore_barrier()
    pltpu.sync_copy(acc_c.at[pl.ds(sid * PT, PT)], out_hbm.at[cid, 0, pl.ds(sid * PT, PT)])
    pltpu.sync_copy(acc_r.at[pl.ds(sid * PT, PT)], out_hbm.at[cid, 1, pl.ds(sid * PT, PT)])


def _segsum_body(hd_hbm, eir_hbm, eic_hbm, z2_hbm, out_hbm, idxr, idxc, rows0,
                 rows1, s_sh, sem0, sem1):
    cid = lax.axis_index("c")
    sid = lax.axis_index("s")
    wid = _wid()
    pltpu.sync_copy(z2_hbm.at[pl.ds(sid * PT, PT)], s_sh.at[pl.ds(sid * PT, PT)])
    plsc.subcore_barrier()

    half_ch = (CH + 1) // 2
    for phase, nch in ((0, half_ch), (1, CH - half_ch)):
        cb = phase * half_ch
        pltpu.sync_copy(eir_hbm.at[wid, pl.ds(cb, nch)], idxr.at[pl.ds(0, nch)])
        pltpu.sync_copy(eic_hbm.at[wid, pl.ds(cb, nch)], idxc.at[pl.ds(0, nch)])
        pltpu.async_copy(hd_hbm.at[idxr.at[0]], rows0, sem0)
        pltpu.async_copy(hd_hbm.at[idxr.at[1]], rows1, sem1)

        def body(i, carry):
            j0 = 2 * i
            j1 = 2 * i + 1
            pltpu.make_async_copy(hd_hbm.at[idxr.at[j0]], rows0, sem0).wait()
            pltpu.sync_copy(rows0, s_sh.at[idxc.at[j0]], add=True)

            @pl.when(j0 + 2 < nch)
            def _():
                pltpu.async_copy(hd_hbm.at[idxr.at[j0 + 2]], rows0, sem0)

            @pl.when(j1 < nch)
            def _():
                pltpu.make_async_copy(hd_hbm.at[idxr.at[j1]], rows1, sem1).wait()
                pltpu.sync_copy(rows1, s_sh.at[idxc.at[j1]], add=True)

                @pl.when(j1 + 2 < nch)
                def _():
                    pltpu.async_copy(hd_hbm.at[idxr.at[j1 + 2]], rows1, sem1)

            return carry

        lax.fori_loop(0, (nch + 1) // 2, body, 0)
    plsc.subcore_barrier()
    pltpu.sync_copy(s_sh.at[pl.ds(sid * PT, PT)], out_hbm.at[cid, pl.ds(sid * PT, PT)])


def _egather_body(g_hbm, eir_hbm, out_hbm, idxr, rows0, rows1, rows64, sem0,
                  sem1):
    wid = _wid()
    base = wid * EPT
    pltpu.sync_copy(eir_hbm.at[wid], idxr)
    nfull = EPT // 128

    def compact(rows, nrows):
        def crow(rr, carry):
            for kk in range(4):
                rows64[rr, pl.ds(kk * 16, 16)] = rows[rr, pl.ds(kk * 16, 16)]
            return carry
        lax.fori_loop(0, nrows, crow, 0)

    pltpu.async_copy(g_hbm.at[idxr.at[0]], rows0, sem0)
    pltpu.async_copy(g_hbm.at[idxr.at[1]], rows1, sem1)

    def half(j, rows, sem):
        pltpu.make_async_copy(g_hbm.at[idxr.at[j]], rows, sem).wait()
        compact(rows, 128)
        pltpu.sync_copy(rows64, out_hbm.at[pl.ds(base + j * 128, 128)])

        @pl.when(j + 2 < CH)
        def _():
            pltpu.async_copy(g_hbm.at[idxr.at[j + 2]], rows, sem)

    def body(i, carry):
        half(2 * i, rows0, sem0)
        half(2 * i + 1, rows1, sem1)
        return carry

    lax.fori_loop(0, nfull // 2, body, 0)
    tail = EPT - nfull * 128
    pltpu.make_async_copy(g_hbm.at[idxr.at[nfull]], rows0, sem0).wait()
    compact(rows0, tail)
    pltpu.sync_copy(rows64.at[pl.ds(0, tail)], out_hbm.at[pl.ds(base + nfull * 128, tail)])


@functools.lru_cache(maxsize=None)
def _sc_kernels():
    mesh = plsc.VectorSubcoreMesh(core_axis_name="c", subcore_axis_name="s",
                                  num_cores=NC, num_subcores=NS)
    hist = pl.kernel(
        _hist_body,
        out_type=jax.ShapeDtypeStruct((NC, 2, NP), jnp.float32),
        mesh=mesh,
        scratch_types=[
            pltpu.VMEM((CH, 128), jnp.int32),
            pltpu.VMEM((CH, 128), jnp.int32),
            pltpu.VMEM((128,), jnp.float32),
            pltpu.VMEM_SHARED((NP,), jnp.float32),
            pltpu.VMEM_SHARED((NP,), jnp.float32),
        ],
    )
    segsum = pl.kernel(
        _segsum_body,
        out_type=jax.ShapeDtypeStruct((NC, NP, 128), jnp.float32),
        mesh=mesh,
        scratch_types=[
            pltpu.VMEM(((CH + 1) // 2, 128), jnp.int32),
            pltpu.VMEM(((CH + 1) // 2, 128), jnp.int32),
            pltpu.VMEM((128, 128), jnp.float32),
            pltpu.VMEM((128, 128), jnp.float32),
            pltpu.VMEM_SHARED((NP, 128), jnp.float32),
            pltpu.SemaphoreType.DMA,
            pltpu.SemaphoreType.DMA,
        ],
    )
    egather = pl.kernel(
        _egather_body,
        out_type=jax.ShapeDtypeStruct((E, 64), jnp.float32),
        mesh=mesh,
        scratch_types=[
            pltpu.VMEM((CH, 128), jnp.int32),
            pltpu.VMEM((128, 128), jnp.float32),
            pltpu.VMEM((128, 128), jnp.float32),
            pltpu.VMEM((128, 64), jnp.float32),
            pltpu.SemaphoreType.DMA,
            pltpu.SemaphoreType.DMA,
        ],
    )
    return hist, segsum, egather


def _dense1_body(x_ref, w_ref, hist_ref, hd_ref, hod_ref):
    h = lax.dot_general(x_ref[...], w_ref[...], (((1,), (1,)), ((), ())),
                        preferred_element_type=jnp.float32)
    cnt = hist_ref[0, 0, :] + hist_ref[1, 0, :]
    dis = lax.rsqrt(cnt + 1.0).reshape(NP, 1)
    hd_ref[...] = h * dis
    hod_ref[...] = h * (dis * dis)


def _dense1(xp, W_mp, hist):
    return pl.pallas_call(
        _dense1_body,
        out_shape=[jax.ShapeDtypeStruct((NP, 128), jnp.float32),
                   jax.ShapeDtypeStruct((NP, 128), jnp.float32)],
    )(xp, W_mp, hist)


def _dense2_body(s_ref, hod_ref, hist_ref, b_ref, wne_ref, we_ref, wen_ref,
                 wrec_ref, g_ref, xr_ref):
    cnt = hist_ref[0, 0, :] + hist_ref[1, 0, :]
    dis = lax.rsqrt(cnt + 1.0).reshape(NP, 1)
    odeg = (hist_ref[0, 1, :] + hist_ref[1, 1, :]).reshape(NP, 1)
    s = s_ref[0] + s_ref[1]
    h_agg = s * dis + hod_ref[...] + b_ref[...]
    w1 = lax.dot_general(we_ref[...], wne_ref[...], (((1,), (0,)), ((), ())),
                         preferred_element_type=jnp.float32)
    g = lax.dot_general(h_agg, w1, (((1,), (1,)), ((), ())),
                        preferred_element_type=jnp.float32)
    g_ref[...] = jnp.concatenate([g, jnp.zeros_like(g)], axis=1)
    wc2 = lax.dot_general(wen_ref[...], w1, (((1,), (0,)), ((), ())),
                          preferred_element_type=jnp.float32)
    w3 = lax.dot_general(wrec_ref[...], wc2, (((1,), (0,)), ((), ())),
                         preferred_element_type=jnp.float32)
    xr_ref[...] = lax.dot_general(h_agg, w3, (((1,), (1,)), ((), ())),
                                  preferred_element_type=jnp.float32) * odeg


def _dense2(s_part, hod, hist, b2, W_ne, W_e, W_en, W_recon):
    return pl.pallas_call(
        _dense2_body,
        out_shape=[jax.ShapeDtypeStruct((NP, 128), jnp.float32),
                   jax.ShapeDtypeStruct((NP, 128), jnp.float32)],
    )(s_part, hod, hist, b2, W_ne, W_e, W_en, W_recon)


def kernel(x, edge_index, W_mp, b_mp, W_ne, W_e, W_en, W_recon):
    n = x.shape[0]
    xp = jnp.pad(x, ((0, NP - n), (0, 0)))
    ei = jnp.pad(edge_index.reshape(2, NW, EPT), ((0, 0), (0, 0), (0, EPTP - EPT)),
                 constant_values=N)
    eir3 = ei[0].reshape(NW, CH, 128)
    eic3 = ei[1].reshape(NW, CH, 128)
    z1 = jnp.zeros((NP,), jnp.float32)
    z2 = jnp.zeros((NP, 128), jnp.float32)

    hist_k, segsum_k, egather_k = _sc_kernels()
    hist = hist_k(eir3, eic3, z1)
    hd, hod = _dense1(xp, W_mp, hist)
    s_part = segsum_k(hd, eir3, eic3, z2)
    g, xr = _dense2(s_part, hod, hist, b_mp.reshape(1, -1), W_ne, W_e, W_en,
                    W_recon)
    ef = egather_k(g, eir3)
    return (xr[:n], ef)


# speedup vs baseline: 13.8701x; 1.0103x over previous
import functools

import jax
import jax.numpy as jnp
from jax import lax
from jax.experimental import pallas as pl
from jax.experimental.pallas import tpu as pltpu
from jax.experimental.pallas import tpu_sc as plsc

N = 10000
NP = 10240
E = 320000
NC = 2
NS = 16
NW = NC * NS
EPT = E // NW
CH = (EPT + 127) // 128
EPTP = CH * 128
PT = NP // NS

def _wid():
    return lax.axis_index("s") * NC + lax.axis_index("c")


def _hist_body(eir_hbm, eic_hbm, z_hbm, out_hbm, idxr, idxc, ones_v, acc_c, acc_r):
    cid = lax.axis_index("c")
    sid = lax.axis_index("s")
    wid = _wid()
    pltpu.sync_copy(z_hbm.at[pl.ds(sid * PT, PT)], acc_c.at[pl.ds(sid * PT, PT)])
    pltpu.sync_copy(z_hbm.at[pl.ds(sid * PT, PT)], acc_r.at[pl.ds(sid * PT, PT)])
    for i in range(8):
        ones_v[pl.ds(i * 16, 16)] = jnp.full((16,), 1.0, jnp.float32)
    pltpu.sync_copy(eir_hbm.at[wid], idxr)
    pltpu.sync_copy(eic_hbm.at[wid], idxc)
    plsc.subcore_barrier()

    def body(j, carry):
        pltpu.sync_copy(ones_v, acc_c.at[idxc.at[j]], add=True)
        pltpu.sync_copy(ones_v, acc_r.at[idxr.at[j]], add=True)
        return carry

    lax.fori_loop(0, CH, body, 0)
    plsc.subcore_barrier()
    pltpu.sync_copy(acc_c.at[pl.ds(sid * PT, PT)], out_hbm.at[cid, 0, pl.ds(sid * PT, PT)])
    pltpu.sync_copy(acc_r.at[pl.ds(sid * PT, PT)], out_hbm.at[cid, 1, pl.ds(sid * PT, PT)])


def _segsum_body(hd_hbm, eir_hbm, eic_hbm, z2_hbm, out_hbm, idxr, idxc, rows0,
                 rows1, s_sh, sem0, sem1):
    cid = lax.axis_index("c")
    sid = lax.axis_index("s")
    wid = _wid()
    pltpu.sync_copy(z2_hbm.at[pl.ds(sid * PT, PT)], s_sh.at[pl.ds(sid * PT, PT)])
    plsc.subcore_barrier()

    half_ch = (CH + 1) // 2
    for phase, nch in ((0, half_ch), (1, CH - half_ch)):
        cb = phase * half_ch
        pltpu.sync_copy(eir_hbm.at[wid, pl.ds(cb, nch)], idxr.at[pl.ds(0, nch)])
        pltpu.sync_copy(eic_hbm.at[wid, pl.ds(cb, nch)], idxc.at[pl.ds(0, nch)])
        pltpu.async_copy(hd_hbm.at[idxr.at[0]], rows0, sem0)
        pltpu.async_copy(hd_hbm.at[idxr.at[1]], rows1, sem1)

        def body(i, carry):
            j0 = 2 * i
            j1 = 2 * i + 1
            pltpu.make_async_copy(hd_hbm.at[idxr.at[j0]], rows0, sem0).wait()
            pltpu.sync_copy(rows0, s_sh.at[idxc.at[j0]], add=True)

            @pl.when(j0 + 2 < nch)
            def _():
                pltpu.async_copy(hd_hbm.at[idxr.at[j0 + 2]], rows0, sem0)

            @pl.when(j1 < nch)
            def _():
                pltpu.make_async_copy(hd_hbm.at[idxr.at[j1]], rows1, sem1).wait()
                pltpu.sync_copy(rows1, s_sh.at[idxc.at[j1]], add=True)

                @pl.when(j1 + 2 < nch)
                def _():
                    pltpu.async_copy(hd_hbm.at[idxr.at[j1 + 2]], rows1, sem1)

            return carry

        lax.fori_loop(0, (nch + 1) // 2, body, 0)
    plsc.subcore_barrier()
    pltpu.sync_copy(s_sh.at[pl.ds(sid * PT, PT)], out_hbm.at[cid, pl.ds(sid * PT, PT)])


def _egather_body(g_hbm, eir_hbm, out_hbm, idxr, rows, r64, gs0, gs1, gs2,
                  ws0, ws1, ws2):
    gs = [gs0, gs1, gs2]
    ws = [ws0, ws1, ws2]
    wid = _wid()
    base = wid * EPT
    pltpu.sync_copy(eir_hbm.at[wid], idxr)

    def issue_gather(j, b):
        pltpu.async_copy(g_hbm.at[idxr.at[j]], rows.at[b], gs[b])

    def wait_gather(j, b):
        pltpu.make_async_copy(g_hbm.at[idxr.at[j]], rows.at[b], gs[b]).wait()

    def wait_write(b):
        pltpu.make_async_copy(r64.at[b], out_hbm.at[pl.ds(base, 128)], ws[b]).wait()

    def compact(b, nrows):
        def crow(rr, carry):
            for kk in range(4):
                r64[b, rr, pl.ds(kk * 16, 16)] = rows[b, rr, pl.ds(kk * 16, 16)]
            return carry
        lax.fori_loop(0, nrows, crow, 0)

    def step(j, b, guard_i=None):
        wait_gather(j, b)
        if guard_i is None:
            wait_write(b)
        else:
            @pl.when(guard_i >= 1)
            def _():
                wait_write(b)
        compact(b, 128)
        pltpu.async_copy(r64.at[b], out_hbm.at[pl.ds(base + j * 128, 128)], ws[b])

    for b in range(3):
        issue_gather(b, b)

    def body(i, carry):
        for t in range(3):
            j = 3 * i + t
            step(j, t, guard_i=i)

            @pl.when(j + 3 < CH)
            def _():
                issue_gather(j + 3, t)

        return carry

    lax.fori_loop(0, (CH - 1) // 3, body, 0)
    tail = EPT - (CH - 1) * 128
    wait_gather(78, 0)
    wait_write(0)
    compact(0, tail)
    pltpu.async_copy(r64.at[0, pl.ds(0, tail)],
                     out_hbm.at[pl.ds(base + (CH - 1) * 128, tail)], ws0)
    wait_write(1)
    wait_write(2)
    pltpu.make_async_copy(r64.at[0, pl.ds(0, tail)],
                          out_hbm.at[pl.ds(base, tail)], ws0).wait()


def _sems(n):
    return [pltpu.SemaphoreType.DMA] * n


@functools.lru_cache(maxsize=None)
def _sc_kernels():
    mesh = plsc.VectorSubcoreMesh(core_axis_name="c", subcore_axis_name="s",
                                  num_cores=NC, num_subcores=NS)
    hist = pl.kernel(
        _hist_body,
        out_type=jax.ShapeDtypeStruct((NC, 2, NP), jnp.float32),
        mesh=mesh,
        scratch_types=[
            pltpu.VMEM((CH, 128), jnp.int32),
            pltpu.VMEM((CH, 128), jnp.int32),
            pltpu.VMEM((128,), jnp.float32),
            pltpu.VMEM_SHARED((NP,), jnp.float32),
            pltpu.VMEM_SHARED((NP,), jnp.float32),
        ],
    )
    segsum = pl.kernel(
        _segsum_body,
        out_type=jax.ShapeDtypeStruct((NC, NP, 128), jnp.float32),
        mesh=mesh,
        scratch_types=[
            pltpu.VMEM(((CH + 1) // 2, 128), jnp.int32),
            pltpu.VMEM(((CH + 1) // 2, 128), jnp.int32),
            pltpu.VMEM((128, 128), jnp.float32),
            pltpu.VMEM((128, 128), jnp.float32),
            pltpu.VMEM_SHARED((NP, 128), jnp.float32),
            pltpu.SemaphoreType.DMA,
            pltpu.SemaphoreType.DMA,
        ],
    )
    egather = pl.kernel(
        _egather_body,
        out_type=jax.ShapeDtypeStruct((E, 64), jnp.float32),
        mesh=mesh,
        scratch_types=[
            pltpu.VMEM((CH, 128), jnp.int32),
            pltpu.VMEM((3, 128, 128), jnp.float32),
            pltpu.VMEM((3, 128, 64), jnp.float32),
        ] + _sems(6),
    )
    return hist, segsum, egather


def _dense1_body(x_ref, w_ref, hist_ref, hd_ref, hod_ref):
    h = lax.dot_general(x_ref[...], w_ref[...], (((1,), (1,)), ((), ())),
                        preferred_element_type=jnp.float32)
    cnt = hist_ref[0, 0, :] + hist_ref[1, 0, :]
    dis = lax.rsqrt(cnt + 1.0).reshape(NP, 1)
    hd_ref[...] = h * dis
    hod_ref[...] = h * (dis * dis)


def _dense1(xp, W_mp, hist):
    return pl.pallas_call(
        _dense1_body,
        out_shape=[jax.ShapeDtypeStruct((NP, 128), jnp.float32),
                   jax.ShapeDtypeStruct((NP, 128), jnp.float32)],
    )(xp, W_mp, hist)


def _dense2_body(s_ref, hod_ref, hist_ref, b_ref, wne_ref, we_ref, wen_ref,
                 wrec_ref, g_ref, xr_ref):
    cnt = hist_ref[0, 0, :] + hist_ref[1, 0, :]
    dis = lax.rsqrt(cnt + 1.0).reshape(NP, 1)
    odeg = (hist_ref[0, 1, :] + hist_ref[1, 1, :]).reshape(NP, 1)
    s = s_ref[0] + s_ref[1]
    h_agg = s * dis + hod_ref[...] + b_ref[...]
    w1 = lax.dot_general(we_ref[...], wne_ref[...], (((1,), (0,)), ((), ())),
                         preferred_element_type=jnp.float32)
    g = lax.dot_general(h_agg, w1, (((1,), (1,)), ((), ())),
                        preferred_element_type=jnp.float32)
    g_ref[...] = jnp.concatenate([g, jnp.zeros_like(g)], axis=1)
    wc2 = lax.dot_general(wen_ref[...], w1, (((1,), (0,)), ((), ())),
                          preferred_element_type=jnp.float32)
    w3 = lax.dot_general(wrec_ref[...], wc2, (((1,), (0,)), ((), ())),
                         preferred_element_type=jnp.float32)
    xr_ref[...] = lax.dot_general(h_agg, w3, (((1,), (1,)), ((), ())),
                                  preferred_element_type=jnp.float32) * odeg


def _dense2(s_part, hod, hist, b2, W_ne, W_e, W_en, W_recon):
    return pl.pallas_call(
        _dense2_body,
        out_shape=[jax.ShapeDtypeStruct((NP, 128), jnp.float32),
                   jax.ShapeDtypeStruct((NP, 128), jnp.float32)],
    )(s_part, hod, hist, b2, W_ne, W_e, W_en, W_recon)


def kernel(x, edge_index, W_mp, b_mp, W_ne, W_e, W_en, W_recon):
    n = x.shape[0]
    xp = jnp.pad(x, ((0, NP - n), (0, 0)))
    ei = jnp.pad(edge_index.reshape(2, NW, EPT), ((0, 0), (0, 0), (0, EPTP - EPT)),
                 constant_values=N)
    eir3 = ei[0].reshape(NW, CH, 128)
    eic3 = ei[1].reshape(NW, CH, 128)
    z1 = jnp.zeros((NP,), jnp.float32)
    z2 = jnp.zeros((NP, 128), jnp.float32)

    hist_k, segsum_k, egather_k = _sc_kernels()
    hist = hist_k(eir3, eic3, z1)
    hd, hod = _dense1(xp, W_mp, hist)
    s_part = segsum_k(hd, eir3, eic3, z2)
    g, xr = _dense2(s_part, hod, hist, b_mp.reshape(1, -1), W_ne, W_e, W_en,
                    W_recon)
    ef = egather_k(g, eir3)
    return (xr[:n], ef)


# speedup vs baseline: 20.0580x; 1.4461x over previous
import functools

import jax
import jax.numpy as jnp
from jax import lax
from jax.experimental import pallas as pl
from jax.experimental.pallas import tpu as pltpu
from jax.experimental.pallas import tpu_sc as plsc

N = 10000
NP = 10240
E = 320000
NC = 2
NS = 16
NW = NC * NS
EPT = E // NW
CH = (EPT + 127) // 128
EPTP = CH * 128
PT = NP // NS

def _wid():
    return lax.axis_index("s") * NC + lax.axis_index("c")


def _hist_body(eir_hbm, eic_hbm, z_hbm, out_hbm, idxr, idxc, ones_v, acc_c, acc_r):
    cid = lax.axis_index("c")
    sid = lax.axis_index("s")
    wid = _wid()
    pltpu.sync_copy(z_hbm.at[pl.ds(sid * PT, PT)], acc_c.at[pl.ds(sid * PT, PT)])
    pltpu.sync_copy(z_hbm.at[pl.ds(sid * PT, PT)], acc_r.at[pl.ds(sid * PT, PT)])
    for i in range(8):
        ones_v[pl.ds(i * 16, 16)] = jnp.full((16,), 1.0, jnp.float32)
    pltpu.sync_copy(eir_hbm.at[wid], idxr)
    pltpu.sync_copy(eic_hbm.at[wid], idxc)
    plsc.subcore_barrier()

    def body(j, carry):
        pltpu.sync_copy(ones_v, acc_c.at[idxc.at[j]], add=True)
        pltpu.sync_copy(ones_v, acc_r.at[idxr.at[j]], add=True)
        return carry

    lax.fori_loop(0, CH, body, 0)
    plsc.subcore_barrier()
    pl.delay(16384)
    pltpu.sync_copy(acc_c.at[pl.ds(sid * PT, PT)], out_hbm.at[cid, 0, pl.ds(sid * PT, PT)])
    pltpu.sync_copy(acc_r.at[pl.ds(sid * PT, PT)], out_hbm.at[cid, 1, pl.ds(sid * PT, PT)])


def _segsum_body(hd_hbm, eir_hbm, eic_hbm, z2_hbm, out_hbm, idxr, idxc, rows0,
                 rows1, s_sh, sem0, sem1):
    cid = lax.axis_index("c")
    sid = lax.axis_index("s")
    wid = _wid()
    pltpu.sync_copy(z2_hbm.at[pl.ds(sid * PT, PT)], s_sh.at[pl.ds(sid * PT, PT)])
    plsc.subcore_barrier()

    half_ch = (CH + 1) // 2
    for phase, nch in ((0, half_ch), (1, CH - half_ch)):
        cb = phase * half_ch
        pltpu.sync_copy(eir_hbm.at[wid, pl.ds(cb, nch)], idxr.at[pl.ds(0, nch)])
        pltpu.sync_copy(eic_hbm.at[wid, pl.ds(cb, nch)], idxc.at[pl.ds(0, nch)])
        pltpu.async_copy(hd_hbm.at[idxr.at[0]], rows0, sem0)
        pltpu.async_copy(hd_hbm.at[idxr.at[1]], rows1, sem1)

        def body(i, carry):
            j0 = 2 * i
            j1 = 2 * i + 1
            pltpu.make_async_copy(hd_hbm.at[idxr.at[j0]], rows0, sem0).wait()
            pltpu.sync_copy(rows0, s_sh.at[idxc.at[j0]], add=True)

            @pl.when(j0 + 2 < nch)
            def _():
                pltpu.async_copy(hd_hbm.at[idxr.at[j0 + 2]], rows0, sem0)

            @pl.when(j1 < nch)
            def _():
                pltpu.make_async_copy(hd_hbm.at[idxr.at[j1]], rows1, sem1).wait()
                pltpu.sync_copy(rows1, s_sh.at[idxc.at[j1]], add=True)

                @pl.when(j1 + 2 < nch)
                def _():
                    pltpu.async_copy(hd_hbm.at[idxr.at[j1 + 2]], rows1, sem1)

            return carry

        lax.fori_loop(0, (nch + 1) // 2, body, 0)
    plsc.subcore_barrier()
    pl.delay(16384)
    pltpu.sync_copy(s_sh.at[pl.ds(sid * PT, PT)], out_hbm.at[cid, pl.ds(sid * PT, PT)])


def _egather_body(g_hbm, eir_hbm, out_hbm, idxr, rows, r64, gs0, gs1, gs2,
                  ws0, ws1, ws2):
    gs = [gs0, gs1, gs2]
    ws = [ws0, ws1, ws2]
    wid = _wid()
    base = wid * EPT
    pltpu.sync_copy(eir_hbm.at[wid], idxr)

    def issue_gather(j, b):
        pltpu.async_copy(g_hbm.at[idxr.at[j]], rows.at[b], gs[b])

    def wait_gather(j, b):
        pltpu.make_async_copy(g_hbm.at[idxr.at[j]], rows.at[b], gs[b]).wait()

    def wait_write(b):
        pltpu.make_async_copy(r64.at[b], out_hbm.at[pl.ds(base, 128)], ws[b]).wait()

    def compact(b, nrows):
        def crow(i8, carry):
            for r8 in range(8):
                for kk in range(4):
                    r64[b, i8 * 8 + r8, pl.ds(kk * 16, 16)] = (
                        rows[b, i8 * 8 + r8, pl.ds(kk * 16, 16)])
            return carry
        lax.fori_loop(0, nrows // 8, crow, 0)

    def step(j, b, guard_i=None):
        wait_gather(j, b)
        if guard_i is None:
            wait_write(b)
        else:
            @pl.when(guard_i >= 1)
            def _():
                wait_write(b)
        compact(b, 128)
        pltpu.async_copy(r64.at[b], out_hbm.at[pl.ds(base + j * 128, 128)], ws[b])

    for b in range(3):
        issue_gather(b, b)

    def body(i, carry):
        for t in range(3):
            j = 3 * i + t
            step(j, t, guard_i=i)

            @pl.when(j + 3 < CH)
            def _():
                issue_gather(j + 3, t)

        return carry

    lax.fori_loop(0, (CH - 1) // 3, body, 0)
    tail = EPT - (CH - 1) * 128
    wait_gather(78, 0)
    wait_write(0)
    compact(0, tail)
    pltpu.async_copy(r64.at[0, pl.ds(0, tail)],
                     out_hbm.at[pl.ds(base + (CH - 1) * 128, tail)], ws0)
    wait_write(1)
    wait_write(2)
    pltpu.make_async_copy(r64.at[0, pl.ds(0, tail)],
                          out_hbm.at[pl.ds(base, tail)], ws0).wait()


def _sems(n):
    return [pltpu.SemaphoreType.DMA] * n


@functools.lru_cache(maxsize=None)
def _sc_kernels():
    mesh = plsc.VectorSubcoreMesh(core_axis_name="c", subcore_axis_name="s",
                                  num_cores=NC, num_subcores=NS)
    hist = pl.kernel(
        _hist_body,
        out_type=jax.ShapeDtypeStruct((NC, 2, NP), jnp.float32),
        mesh=mesh,
        scratch_types=[
            pltpu.VMEM((CH, 128), jnp.int32),
            pltpu.VMEM((CH, 128), jnp.int32),
            pltpu.VMEM((128,), jnp.float32),
            pltpu.VMEM_SHARED((NP,), jnp.float32),
            pltpu.VMEM_SHARED((NP,), jnp.float32),
        ],
    )
    segsum = pl.kernel(
        _segsum_body,
        out_type=jax.ShapeDtypeStruct((NC, NP, 128), jnp.float32),
        mesh=mesh,
        scratch_types=[
            pltpu.VMEM(((CH + 1) // 2, 128), jnp.int32),
            pltpu.VMEM(((CH + 1) // 2, 128), jnp.int32),
            pltpu.VMEM((128, 128), jnp.float32),
            pltpu.VMEM((128, 128), jnp.float32),
            pltpu.VMEM_SHARED((NP, 128), jnp.float32),
            pltpu.SemaphoreType.DMA,
            pltpu.SemaphoreType.DMA,
        ],
    )
    egather = pl.kernel(
        _egather_body,
        out_type=jax.ShapeDtypeStruct((E, 64), jnp.float32),
        mesh=mesh,
        scratch_types=[
            pltpu.VMEM((CH, 128), jnp.int32),
            pltpu.VMEM((3, 128, 128), jnp.float32),
            pltpu.VMEM((3, 128, 64), jnp.float32),
        ] + _sems(6),
    )
    return hist, segsum, egather


def _dense1_body(x_ref, w_ref, hist_ref, hd_ref, hod_ref):
    h = lax.dot_general(x_ref[...], w_ref[...], (((1,), (1,)), ((), ())),
                        preferred_element_type=jnp.float32)
    cnt = hist_ref[0, 0, :] + hist_ref[1, 0, :]
    dis = lax.rsqrt(cnt + 1.0).reshape(NP, 1)
    hd_ref[...] = h * dis
    hod_ref[...] = h * (dis * dis)


def _dense1(xp, W_mp, hist):
    return pl.pallas_call(
        _dense1_body,
        out_shape=[jax.ShapeDtypeStruct((NP, 128), jnp.float32),
                   jax.ShapeDtypeStruct((NP, 128), jnp.float32)],
    )(xp, W_mp, hist)


def _dense2_body(s_ref, hod_ref, hist_ref, b_ref, wne_ref, we_ref, wen_ref,
                 wrec_ref, g_ref, xr_ref):
    cnt = hist_ref[0, 0, :] + hist_ref[1, 0, :]
    dis = lax.rsqrt(cnt + 1.0).reshape(NP, 1)
    odeg = (hist_ref[0, 1, :] + hist_ref[1, 1, :]).reshape(NP, 1)
    s = s_ref[0] + s_ref[1]
    h_agg = s * dis + hod_ref[...] + b_ref[...]
    w1 = lax.dot_general(we_ref[...], wne_ref[...], (((1,), (0,)), ((), ())),
                         preferred_element_type=jnp.float32)
    g = lax.dot_general(h_agg, w1, (((1,), (1,)), ((), ())),
                        preferred_element_type=jnp.float32)
    g_ref[...] = jnp.concatenate([g, jnp.zeros_like(g)], axis=1)
    wc2 = lax.dot_general(wen_ref[...], w1, (((1,), (0,)), ((), ())),
                          preferred_element_type=jnp.float32)
    w3 = lax.dot_general(wrec_ref[...], wc2, (((1,), (0,)), ((), ())),
                         preferred_element_type=jnp.float32)
    xr_ref[...] = lax.dot_general(h_agg, w3, (((1,), (1,)), ((), ())),
                                  preferred_element_type=jnp.float32) * odeg


def _dense2(s_part, hod, hist, b2, W_ne, W_e, W_en, W_recon):
    return pl.pallas_call(
        _dense2_body,
        out_shape=[jax.ShapeDtypeStruct((NP, 128), jnp.float32),
                   jax.ShapeDtypeStruct((NP, 128), jnp.float32)],
    )(s_part, hod, hist, b2, W_ne, W_e, W_en, W_recon)


def kernel(x, edge_index, W_mp, b_mp, W_ne, W_e, W_en, W_recon):
    n = x.shape[0]
    xp = jnp.pad(x, ((0, NP - n), (0, 0)))
    ei = jnp.pad(edge_index.reshape(2, NW, EPT), ((0, 0), (0, 0), (0, EPTP - EPT)),
                 constant_values=N)
    dump = jnp.broadcast_to((N + jnp.arange(NW, dtype=edge_index.dtype))[:, None],
                            (NW, EPTP - EPT))
    eir3 = ei[0].at[:, EPT:].set(dump).reshape(NW, CH, 128)
    eic3 = ei[1].at[:, EPT:].set(dump).reshape(NW, CH, 128)
    z1 = jnp.zeros((NP,), jnp.float32)
    z2 = jnp.zeros((NP, 128), jnp.float32)

    hist_k, segsum_k, egather_k = _sc_kernels()
    hist = hist_k(eir3, eic3, z1)
    hd, hod = _dense1(xp, W_mp, hist)
    s_part = segsum_k(hd, eir3, eic3, z2)
    g, xr = _dense2(s_part, hod, hist, b_mp.reshape(1, -1), W_ne, W_e, W_en,
                    W_recon)
    ef = egather_k(g, eir3)
    return (xr[:n], ef)
